# baseline (speedup 1.0000x reference)
import functools
import jax
import jax.numpy as jnp
import numpy as np
from jax import lax
from jax.experimental import pallas as pl
from jax.experimental.pallas import tpu as pltpu
from jax.experimental.pallas import tpu_sc as plsc

N = 2048
F = 128
CH = 128
NOSC = 4
NG = CH // NOSC
H = 8
DH = CH // H
QSTEPS = 4
E = 32768
GST = 4
NCLS = 4
GAMMA = 1.0


def _encoder_body(x_ref, Wm_ref, bm_ref, WpyT_ref, bpy_ref, Wout_ref, bout_ref,
                  y_ref, logits_ref):
    x = x_ref[...]
    y = jnp.zeros((N, CH), jnp.float32) + bpy_ref[...][None, :]
    for k in range(GST):
        ys = jnp.maximum(
            jax.lax.dot_general(x, Wm_ref[k], (((1,), (0,)), ((), ())),
                                preferred_element_type=jnp.float32)
            + bm_ref[k][None, :], 0.0)
        y = y + jax.lax.dot_general(ys, WpyT_ref[pl.ds(k * F, F), :],
                                    (((1,), (0,)), ((), ())),
                                    preferred_element_type=jnp.float32)
    y_ref[...] = y
    logits_ref[...] = jax.lax.dot_general(
        y, Wout_ref[...], (((1,), (0,)), ((), ())),
        preferred_element_type=jnp.float32) + bout_ref[...][None, :]


def _encoder(x, Wm, bm, WpyT, bpy, Wout, bout):
    return pl.pallas_call(
        _encoder_body,
        out_shape=(jax.ShapeDtypeStruct((N, CH), jnp.float32),
                   jax.ShapeDtypeStruct((N, NCLS), jnp.float32)),
    )(x, Wm, bm, WpyT, bpy, Wout, bout)


TILES = 16
NPT = N // TILES
EPT = E // TILES
ECH = 128
NCHUNK = EPT // ECH


def _sc_diffuse_body(x0_hbm, src_hbm, dst_hbm,
                     d1_hbm, d2_hbm, d4_hbm, d8_hbm, zn_hbm,
                     z_l, rows_l, zer_l, src_l, dst_l, invdeg_l, acc_sh, sem):
    sid = lax.axis_index("s")
    base = sid * NPT
    ebase = sid * NCHUNK

    pltpu.sync_copy(src_hbm.at[pl.ds(ebase, NCHUNK)], src_l)
    pltpu.sync_copy(dst_hbm.at[pl.ds(ebase, NCHUNK)], dst_l)
    pltpu.sync_copy(x0_hbm.at[pl.ds(base, NPT)], z_l)

    zero16 = jnp.zeros((16,), jnp.float32)
    one16 = jnp.ones((16,), jnp.float32)

    def _fill(ref, val):
        def row(i, _):
            def col(j, _):
                ref[i, pl.ds(j * 16, 16)] = val
                return 0
            return lax.fori_loop(0, F // 16, col, 0)
        lax.fori_loop(0, NPT, row, 0)

    _fill(zer_l, zero16)
    _fill(rows_l, one16)

    pltpu.sync_copy(zer_l, acc_sh.at[pl.ds(base, NPT)])
    plsc.subcore_barrier()

    def deg_chunk(c, _):
        pltpu.sync_copy(rows_l, acc_sh.at[dst_l.at[c]], add=True)
        return 0
    lax.fori_loop(0, NCHUNK, deg_chunk, 0)
    plsc.subcore_barrier()

    pltpu.sync_copy(acc_sh.at[pl.ds(base, NPT)], rows_l)

    def inv_row(i, _):
        d = rows_l[i, pl.ds(0, 16)]
        invdeg_l[i, pl.ds(0, 16)] = 1.0 / jnp.maximum(d, 1.0)
        return 0
    lax.fori_loop(0, NPT, inv_row, 0)

    for step in range(1, 9):
        def zn_row(i, _):
            sv = invdeg_l[i, pl.ds(0, 16)]
            def col(j, _):
                rows_l[i, pl.ds(j * 16, 16)] = z_l[i, pl.ds(j * 16, 16)] * sv
                return 0
            return lax.fori_loop(0, F // 16, col, 0)
        lax.fori_loop(0, NPT, zn_row, 0)
        pltpu.sync_copy(rows_l, zn_hbm.at[pl.ds(base, NPT)])
        pltpu.sync_copy(zer_l, acc_sh.at[pl.ds(base, NPT)])
        plsc.subcore_barrier()

        def edge_chunk(c, _):
            pltpu.async_copy(zn_hbm.at[dst_l.at[c]], rows_l, sem).wait()
            pltpu.sync_copy(rows_l, acc_sh.at[src_l.at[c]], add=True)
            return 0
        lax.fori_loop(0, NCHUNK, edge_chunk, 0)
        plsc.subcore_barrier()

        pltpu.sync_copy(acc_sh.at[pl.ds(base, NPT)], rows_l)

        def upd_row(i, _):
            def col(j, _):
                sl = pl.ds(j * 16, 16)
                z_l[i, sl] = 0.5 * z_l[i, sl] + 0.5 * rows_l[i, sl]
                return 0
            return lax.fori_loop(0, F // 16, col, 0)
        lax.fori_loop(0, NPT, upd_row, 0)

        out_ref = {1: d1_hbm, 2: d2_hbm, 4: d4_hbm, 8: d8_hbm}.get(step)
        if out_ref is not None:
            pltpu.sync_copy(z_l, out_ref.at[pl.ds(base, NPT)])


def _sc_diffuse(x0, src2, dst2):
    mesh = plsc.VectorSubcoreMesh(core_axis_name="c", subcore_axis_name="s",
                                  num_cores=1)
    f = pl.kernel(
        _sc_diffuse_body, mesh=mesh,
        out_type=[jax.ShapeDtypeStruct((N, F), jnp.float32)] * 5,
        scratch_types=[
            pltpu.VMEM((NPT, F), jnp.float32),
            pltpu.VMEM((NPT, F), jnp.float32),
            pltpu.VMEM((NPT, F), jnp.float32),
            pltpu.VMEM((NCHUNK, ECH), jnp.int32),
            pltpu.VMEM((NCHUNK, ECH), jnp.int32),
            pltpu.VMEM((NPT, 16), jnp.float32),
            pltpu.VMEM_SHARED((N, F), jnp.float32),
            pltpu.SemaphoreType.DMA,
        ])
    return f(x0, src2, dst2)


def _kuramoto_body(d1_ref, d2_ref, d4_ref, d8_ref, yt_ref, WpxT_ref, bpx_ref,
                   Wq_ref, Wk_ref, Wv_ref, Wo_ref, Gm_ref, WroS_ref, bro_ref,
                   xout_ref):
    d1 = d1_ref[...]
    d2 = d2_ref[...]
    d4 = d4_ref[...]
    d8 = d8_ref[...]
    x0 = bpx_ref[...][None, :]
    for g, blk in enumerate((d8, jnp.abs(d1 - d2), jnp.abs(d2 - d4),
                             jnp.abs(d4 - d8))):
        x0 = x0 + jax.lax.dot_general(
            blk, WpxT_ref[pl.ds(g * F, F), :], (((1,), (0,)), ((), ())),
            preferred_element_type=jnp.float32)
    Gm = Gm_ref[...]

    def gsum(v):
        return jax.lax.dot_general(v, Gm, (((1,), (0,)), ((), ())),
                                   preferred_element_type=jnp.float32)

    x = x0 * jax.lax.rsqrt(gsum(x0 * x0) + 1e-6)
    yt = yt_ref[...]
    scale = 1.0 / np.sqrt(DH)
    for _ in range(QSTEPS):
        Q = jax.lax.dot_general(x, Wq_ref[...], (((1,), (0,)), ((), ())),
                                preferred_element_type=jnp.float32)
        K = jax.lax.dot_general(x, Wk_ref[...], (((1,), (0,)), ((), ())),
                                preferred_element_type=jnp.float32)
        V = jax.lax.dot_general(x, Wv_ref[...], (((1,), (0,)), ((), ())),
                                preferred_element_type=jnp.float32)
        outs = []
        for h in range(H):
            Qh = Q[:, h * DH:(h + 1) * DH]
            Kh = K[:, h * DH:(h + 1) * DH]
            Vh = V[:, h * DH:(h + 1) * DH]
            S = jax.lax.dot_general(Qh, Kh, (((1,), (1,)), ((), ())),
                                    preferred_element_type=jnp.float32) * scale
            S = S - jnp.max(S, axis=-1, keepdims=True)
            Ex = jnp.exp(S)
            A = Ex / jnp.sum(Ex, axis=-1, keepdims=True)
            outs.append(jax.lax.dot_general(A, Vh, (((1,), (0,)), ((), ())),
                                            preferred_element_type=jnp.float32))
        O = jnp.concatenate(outs, axis=1)
        Jx = jax.lax.dot_general(O, Wo_ref[...], (((1,), (0,)), ((), ())),
                                 preferred_element_type=jnp.float32)
        force = Jx + yt
        dot = gsum(force * x)
        xg = x + GAMMA * (force - dot * x)
        x = xg * jax.lax.rsqrt(gsum(xg * xg) + 1e-6)
    acc = jnp.full((N, CH), 1e-6, jnp.float32)
    for o in range(NOSC):
        zo = jax.lax.dot_general(x, WroS_ref[o], (((1,), (0,)), ((), ())),
                                 preferred_element_type=jnp.float32)
        acc = acc + zo * zo
    xout_ref[...] = jnp.sqrt(acc) + bro_ref[...][None, :]


def _kuramoto(d1, d2, d4, d8, yt, WpxT, bpx, Wq, Wk, Wv, Wo, Gm, WroS, bro):
    return pl.pallas_call(
        _kuramoto_body,
        out_shape=jax.ShapeDtypeStruct((N, CH), jnp.float32),
    )(d1, d2, d4, d8, yt, WpxT, bpx, Wq, Wk, Wv, Wo, Gm, WroS, bro)


def kernel(input, input_fc, input_sc, Wm, bm, Wpy, bpy, Wpx, bpx, Wq, Wk, Wv,
           Wo, Wro, bro, Wout, bout):
    del input_fc
    x = input[0]
    src = input_sc[0].astype(jnp.int32)
    dst = input_sc[1].astype(jnp.int32)

    y_t, logits = _encoder(x, Wm, bm, Wpy.T, bpy, Wout, bout)

    src2 = src.reshape(E // ECH, ECH)
    dst2 = dst.reshape(E // ECH, ECH)
    d1, d2, d4, d8, _zn = _sc_diffuse(x, src2, dst2)

    Gm = jnp.repeat(jnp.repeat(jnp.eye(NG, dtype=jnp.float32), NOSC, axis=0),
                    NOSC, axis=1)
    WroS = jnp.stack([Wro[:, o::NOSC] for o in range(NOSC)], axis=0)
    x_out = _kuramoto(d1, d2, d4, d8, y_t, Wpx.T, bpx, Wq, Wk, Wv, Wo, Gm,
                      WroS, bro)

    logits_out = logits[None, :, :]
    x_out = x_out[None, :, :]
    saved_y = y_t.T[None, :, :]
    return logits_out, x_out, saved_y


# speedup vs baseline: 1.3818x; 1.3818x over previous
import functools
import jax
import jax.numpy as jnp
import numpy as np
from jax import lax
from jax.experimental import pallas as pl
from jax.experimental.pallas import tpu as pltpu
from jax.experimental.pallas import tpu_sc as plsc

N = 2048
F = 128
CH = 128
NOSC = 4
NG = CH // NOSC
H = 8
DH = CH // H
QSTEPS = 4
E = 32768
GST = 4
NCLS = 4
GAMMA = 1.0


def _encoder_body(x_ref, Wm_ref, bm_ref, WpyT_ref, bpy_ref, Wout_ref, bout_ref,
                  y_ref, logits_ref):
    x = x_ref[...]
    y = jnp.zeros((N, CH), jnp.float32) + bpy_ref[...][None, :]
    for k in range(GST):
        ys = jnp.maximum(
            jax.lax.dot_general(x, Wm_ref[k], (((1,), (0,)), ((), ())),
                                preferred_element_type=jnp.float32)
            + bm_ref[k][None, :], 0.0)
        y = y + jax.lax.dot_general(ys, WpyT_ref[pl.ds(k * F, F), :],
                                    (((1,), (0,)), ((), ())),
                                    preferred_element_type=jnp.float32)
    y_ref[...] = y
    logits_ref[...] = jax.lax.dot_general(
        y, Wout_ref[...], (((1,), (0,)), ((), ())),
        preferred_element_type=jnp.float32) + bout_ref[...][None, :]


def _encoder(x, Wm, bm, WpyT, bpy, Wout, bout):
    return pl.pallas_call(
        _encoder_body,
        out_shape=(jax.ShapeDtypeStruct((N, CH), jnp.float32),
                   jax.ShapeDtypeStruct((N, NCLS), jnp.float32)),
    )(x, Wm, bm, WpyT, bpy, Wout, bout)


TILES = 16
NPT = N // TILES
EPT = E // TILES
ECH = 128
NCHUNK = EPT // ECH


def _sc_diffuse_body(x0_hbm, src_hbm, dst_hbm, zout_hbm, w_hbm,
                     w_l, rows_a, rows_b, zer_l, src_l, dst_l,
                     degv_l, invdeg_l, acc_sh, sem_a, sem_b):
    sid = lax.axis_index("s")
    base = sid * NPT
    ebase = sid * NCHUNK

    pltpu.sync_copy(src_hbm.at[pl.ds(ebase, NCHUNK)], src_l)
    pltpu.sync_copy(dst_hbm.at[pl.ds(ebase, NCHUNK)], dst_l)
    pltpu.sync_copy(x0_hbm.at[pl.ds(base, NPT)], w_l)

    zero16 = jnp.zeros((16,), jnp.float32)
    one16 = jnp.ones((16,), jnp.float32)

    def rows_loop(body):
        def outer(i, _):
            body(i)
            return 0
        lax.fori_loop(0, NPT, outer, 0)

    def fill(ref, val):
        def body(i):
            for j in range(F // 16):
                ref[i, pl.ds(j * 16, 16)] = val
        rows_loop(body)

    fill(zer_l, zero16)
    fill(rows_a, one16)

    pltpu.sync_copy(zer_l, acc_sh.at[pl.ds(base, NPT)])
    plsc.subcore_barrier()
    for c in range(NCHUNK):
        pltpu.sync_copy(rows_a, acc_sh.at[dst_l.at[c]], add=True)
    plsc.subcore_barrier()
    pltpu.sync_copy(acc_sh.at[pl.ds(base, NPT)], rows_b)

    def dv_body(i):
        d = rows_b[i, pl.ds(0, 16)]
        dm = jnp.maximum(d, 1.0)
        degv_l[i, pl.ds(0, 16)] = dm
        invdeg_l[i, pl.ds(0, 16)] = 1.0 / dm
    rows_loop(dv_body)

    def scale_body(i):
        iv = invdeg_l[i, pl.ds(0, 16)]
        for j in range(F // 16):
            sl = pl.ds(j * 16, 16)
            w_l[i, sl] = w_l[i, sl] * iv
    rows_loop(scale_body)

    def step_body(s, _):
        pltpu.sync_copy(w_l, w_hbm.at[pl.ds(base, NPT)])
        pltpu.sync_copy(zer_l, acc_sh.at[pl.ds(base, NPT)])
        plsc.subcore_barrier()
        bufs = (rows_a, rows_b)
        sems = (sem_a, sem_b)
        desc = pltpu.async_copy(w_hbm.at[dst_l.at[0]], rows_a, sem_a)
        for c in range(NCHUNK):
            desc.wait()
            if c + 1 < NCHUNK:
                desc = pltpu.async_copy(w_hbm.at[dst_l.at[c + 1]],
                                        bufs[(c + 1) % 2], sems[(c + 1) % 2])
            pltpu.sync_copy(bufs[c % 2], acc_sh.at[src_l.at[c]], add=True)
        plsc.subcore_barrier()
        pltpu.sync_copy(acc_sh.at[pl.ds(base, NPT)], rows_b)

        def upd_body(i):
            iv = invdeg_l[i, pl.ds(0, 16)]
            dg = degv_l[i, pl.ds(0, 16)]
            for j in range(F // 16):
                sl = pl.ds(j * 16, 16)
                wv = 0.5 * w_l[i, sl] + (0.5 * iv) * rows_b[i, sl]
                w_l[i, sl] = wv
                rows_a[i, sl] = wv * dg
        rows_loop(upd_body)
        pltpu.sync_copy(rows_a, zout_hbm.at[s, pl.ds(base, NPT)])
        return 0
    lax.fori_loop(0, 8, step_body, 0)


def _sc_diffuse(x0, src2, dst2):
    mesh = plsc.VectorSubcoreMesh(core_axis_name="c", subcore_axis_name="s",
                                  num_cores=1)
    f = pl.kernel(
        _sc_diffuse_body, mesh=mesh,
        out_type=(jax.ShapeDtypeStruct((8, N, F), jnp.float32),
                  jax.ShapeDtypeStruct((N, F), jnp.float32)),
        scratch_types=[
            pltpu.VMEM((NPT, F), jnp.float32),
            pltpu.VMEM((NPT, F), jnp.float32),
            pltpu.VMEM((NPT, F), jnp.float32),
            pltpu.VMEM((NPT, F), jnp.float32),
            pltpu.VMEM((NCHUNK, ECH), jnp.int32),
            pltpu.VMEM((NCHUNK, ECH), jnp.int32),
            pltpu.VMEM((NPT, 16), jnp.float32),
            pltpu.VMEM((NPT, 16), jnp.float32),
            pltpu.VMEM_SHARED((N, F), jnp.float32),
            pltpu.SemaphoreType.DMA,
            pltpu.SemaphoreType.DMA,
        ])
    return f(x0, src2, dst2)


def _kuramoto_body(d1_ref, d2_ref, d4_ref, d8_ref, yt_ref, WpxT_ref, bpx_ref,
                   Wq_ref, Wk_ref, Wv_ref, Wo_ref, Gm_ref, WroS_ref, bro_ref,
                   xout_ref):
    d1 = d1_ref[...]
    d2 = d2_ref[...]
    d4 = d4_ref[...]
    d8 = d8_ref[...]
    x0 = bpx_ref[...][None, :]
    for g, blk in enumerate((d8, jnp.abs(d1 - d2), jnp.abs(d2 - d4),
                             jnp.abs(d4 - d8))):
        x0 = x0 + jax.lax.dot_general(
            blk, WpxT_ref[pl.ds(g * F, F), :], (((1,), (0,)), ((), ())),
            preferred_element_type=jnp.float32)
    Gm = Gm_ref[...]

    def gsum(v):
        return jax.lax.dot_general(v, Gm, (((1,), (0,)), ((), ())),
                                   preferred_element_type=jnp.float32)

    x = x0 * jax.lax.rsqrt(gsum(x0 * x0) + 1e-6)
    yt = yt_ref[...]
    scale = 1.0 / np.sqrt(DH)
    for _ in range(QSTEPS):
        Q = jax.lax.dot_general(x, Wq_ref[...], (((1,), (0,)), ((), ())),
                                preferred_element_type=jnp.float32)
        K = jax.lax.dot_general(x, Wk_ref[...], (((1,), (0,)), ((), ())),
                                preferred_element_type=jnp.float32)
        V = jax.lax.dot_general(x, Wv_ref[...], (((1,), (0,)), ((), ())),
                                preferred_element_type=jnp.float32)
        ones_col = jnp.ones((N, 1), jnp.float32)
        outs = []
        for h in range(H):
            Qh = Q[:, h * DH:(h + 1) * DH]
            Kh = K[:, h * DH:(h + 1) * DH]
            Vh1 = jnp.concatenate([V[:, h * DH:(h + 1) * DH], ones_col],
                                  axis=1)
            S = jax.lax.dot_general(Qh, Kh, (((1,), (1,)), ((), ())),
                                    preferred_element_type=jnp.float32) * scale
            Ex = jnp.exp(S)
            O2 = jax.lax.dot_general(Ex, Vh1, (((1,), (0,)), ((), ())),
                                     preferred_element_type=jnp.float32)
            outs.append(O2[:, :DH] / O2[:, DH:DH + 1])
        O = jnp.concatenate(outs, axis=1)
        Jx = jax.lax.dot_general(O, Wo_ref[...], (((1,), (0,)), ((), ())),
                                 preferred_element_type=jnp.float32)
        force = Jx + yt
        dot = gsum(force * x)
        xg = x + GAMMA * (force - dot * x)
        x = xg * jax.lax.rsqrt(gsum(xg * xg) + 1e-6)
    acc = jnp.full((N, CH), 1e-6, jnp.float32)
    for o in range(NOSC):
        zo = jax.lax.dot_general(x, WroS_ref[o], (((1,), (0,)), ((), ())),
                                 preferred_element_type=jnp.float32)
        acc = acc + zo * zo
    xout_ref[...] = jnp.sqrt(acc) + bro_ref[...][None, :]


def _kuramoto(d1, d2, d4, d8, yt, WpxT, bpx, Wq, Wk, Wv, Wo, Gm, WroS, bro):
    return pl.pallas_call(
        _kuramoto_body,
        out_shape=jax.ShapeDtypeStruct((N, CH), jnp.float32),
    )(d1, d2, d4, d8, yt, WpxT, bpx, Wq, Wk, Wv, Wo, Gm, WroS, bro)


def kernel(input, input_fc, input_sc, Wm, bm, Wpy, bpy, Wpx, bpx, Wq, Wk, Wv,
           Wo, Wro, bro, Wout, bout):
    del input_fc
    x = input[0]
    src = input_sc[0].astype(jnp.int32)
    dst = input_sc[1].astype(jnp.int32)

    y_t, logits = _encoder(x, Wm, bm, Wpy.T, bpy, Wout, bout)

    src2 = src.reshape(E // ECH, ECH)
    dst2 = dst.reshape(E // ECH, ECH)
    zout, _w = _sc_diffuse(x, src2, dst2)
    d1, d2, d4, d8 = zout[0], zout[1], zout[3], zout[7]

    Gm = jnp.repeat(jnp.repeat(jnp.eye(NG, dtype=jnp.float32), NOSC, axis=0),
                    NOSC, axis=1)
    WroS = jnp.stack([Wro[:, o::NOSC] for o in range(NOSC)], axis=0)
    x_out = _kuramoto(d1, d2, d4, d8, y_t, Wpx.T, bpx, Wq, Wk, Wv, Wo, Gm,
                      WroS, bro)

    logits_out = logits[None, :, :]
    x_out = x_out[None, :, :]
    saved_y = y_t.T[None, :, :]
    return logits_out, x_out, saved_y


# speedup vs baseline: 1.3875x; 1.0041x over previous
import functools
import jax
import jax.numpy as jnp
import numpy as np
from jax import lax
from jax.experimental import pallas as pl
from jax.experimental.pallas import tpu as pltpu
from jax.experimental.pallas import tpu_sc as plsc

N = 2048
F = 128
CH = 128
NOSC = 4
NG = CH // NOSC
H = 8
DH = CH // H
QSTEPS = 4
E = 32768
GST = 4
NCLS = 4
GAMMA = 1.0


def _encoder_body(x_ref, Wm_ref, bm_ref, WpyT_ref, bpy_ref, Wout_ref, bout_ref,
                  y_ref, logits_ref):
    x = x_ref[...]
    y = jnp.zeros((N, CH), jnp.float32) + bpy_ref[...][None, :]
    for k in range(GST):
        ys = jnp.maximum(
            jax.lax.dot_general(x, Wm_ref[k], (((1,), (0,)), ((), ())),
                                preferred_element_type=jnp.float32)
            + bm_ref[k][None, :], 0.0)
        y = y + jax.lax.dot_general(ys, WpyT_ref[pl.ds(k * F, F), :],
                                    (((1,), (0,)), ((), ())),
                                    preferred_element_type=jnp.float32)
    y_ref[...] = y
    logits_ref[...] = jax.lax.dot_general(
        y, Wout_ref[...], (((1,), (0,)), ((), ())),
        preferred_element_type=jnp.float32) + bout_ref[...][None, :]


def _encoder(x, Wm, bm, WpyT, bpy, Wout, bout):
    return pl.pallas_call(
        _encoder_body,
        out_shape=(jax.ShapeDtypeStruct((N, CH), jnp.float32),
                   jax.ShapeDtypeStruct((N, NCLS), jnp.float32)),
    )(x, Wm, bm, WpyT, bpy, Wout, bout)


TILES = 16
NPT = N // TILES
EPT = E // TILES
ECH = 128
NCHUNK = EPT // ECH


def _sc_diffuse_body(x0_hbm, src_hbm, dst_hbm, zout_hbm, w_hbm,
                     w_l, rows_a, rows_b, rows_c, zer_l,
                     src_l, dst_l, invdeg_l, acc_sh,
                     sem_a, sem_b, sem_c, sem_p):
    sid = lax.axis_index("s")
    base = sid * NPT
    ebase = sid * NCHUNK

    pltpu.sync_copy(src_hbm.at[pl.ds(ebase, NCHUNK)], src_l)
    pltpu.sync_copy(dst_hbm.at[pl.ds(ebase, NCHUNK)], dst_l)
    pltpu.sync_copy(x0_hbm.at[pl.ds(base, NPT)], w_l)

    zero16 = jnp.zeros((16,), jnp.float32)
    one16 = jnp.ones((16,), jnp.float32)

    def rows_loop(body):
        def outer(i, _):
            body(i)
            return 0
        lax.fori_loop(0, NPT, outer, 0)

    def fill(ref, val):
        def body(i):
            for j in range(F // 16):
                ref[i, pl.ds(j * 16, 16)] = val
        rows_loop(body)

    fill(zer_l, zero16)
    fill(rows_a, one16)

    pltpu.sync_copy(zer_l, acc_sh.at[pl.ds(base, NPT)])
    plsc.subcore_barrier()
    for c in range(NCHUNK):
        pltpu.sync_copy(rows_a, acc_sh.at[dst_l.at[c]], add=True)
    plsc.subcore_barrier()
    pltpu.sync_copy(acc_sh.at[pl.ds(base, NPT)], rows_b)

    def iv_slot(i):
        return (i // 8, pl.ds((i % 8) * 16, 16))

    def dv_body(i):
        d = rows_b[i, pl.ds(0, 16)]
        r, sl = iv_slot(i)
        invdeg_l[r, sl] = 1.0 / jnp.maximum(d, 1.0)
    rows_loop(dv_body)

    def scale_body(i):
        r, s0 = iv_slot(i)
        iv = invdeg_l[r, s0]
        for j in range(F // 16):
            sl = pl.ds(j * 16, 16)
            w_l[i, sl] = w_l[i, sl] * iv
    rows_loop(scale_body)

    def step_body(s, _):
        pub = pltpu.async_copy(w_l, w_hbm.at[pl.ds(base, NPT)], sem_p)
        pltpu.sync_copy(zer_l, acc_sh.at[pl.ds(base, NPT)])
        pub.wait()
        plsc.subcore_barrier()
        bufs = (rows_a, rows_b, rows_c)
        sems = (sem_a, sem_b, sem_c)
        descs = [None, None, None]
        descs[0] = pltpu.async_copy(w_hbm.at[dst_l.at[0]], rows_a, sem_a)
        descs[1] = pltpu.async_copy(w_hbm.at[dst_l.at[1]], rows_b, sem_b)
        for c in range(NCHUNK):
            descs[c % 3].wait()
            if c + 2 < NCHUNK:
                descs[(c + 2) % 3] = pltpu.async_copy(
                    w_hbm.at[dst_l.at[c + 2]], bufs[(c + 2) % 3],
                    sems[(c + 2) % 3])
            pltpu.sync_copy(bufs[c % 3], acc_sh.at[src_l.at[c]], add=True)
        plsc.subcore_barrier()
        pltpu.sync_copy(acc_sh.at[pl.ds(base, NPT)], rows_b)

        def upd_body(i):
            r, s0 = iv_slot(i)
            iv = invdeg_l[r, s0]
            for j in range(F // 16):
                sl = pl.ds(j * 16, 16)
                wv = 0.5 * w_l[i, sl] + (0.5 * iv) * rows_b[i, sl]
                w_l[i, sl] = wv
                rows_a[i, sl] = wv / iv
        rows_loop(upd_body)
        pltpu.sync_copy(rows_a, zout_hbm.at[s, pl.ds(base, NPT)])
        return 0
    lax.fori_loop(0, 8, step_body, 0)


def _sc_diffuse(x0, src2, dst2):
    mesh = plsc.VectorSubcoreMesh(core_axis_name="c", subcore_axis_name="s",
                                  num_cores=1)
    f = pl.kernel(
        _sc_diffuse_body, mesh=mesh,
        out_type=(jax.ShapeDtypeStruct((8, N, F), jnp.float32),
                  jax.ShapeDtypeStruct((N, F), jnp.float32)),
        scratch_types=[
            pltpu.VMEM((NPT, F), jnp.float32),
            pltpu.VMEM((NPT, F), jnp.float32),
            pltpu.VMEM((NPT, F), jnp.float32),
            pltpu.VMEM((NPT, F), jnp.float32),
            pltpu.VMEM((NPT, F), jnp.float32),
            pltpu.VMEM((NCHUNK, ECH), jnp.int32),
            pltpu.VMEM((NCHUNK, ECH), jnp.int32),
            pltpu.VMEM((NPT // 8, 128), jnp.float32),
            pltpu.VMEM_SHARED((N, F), jnp.float32),
            pltpu.SemaphoreType.DMA,
            pltpu.SemaphoreType.DMA,
            pltpu.SemaphoreType.DMA,
            pltpu.SemaphoreType.DMA,
        ])
    return f(x0, src2, dst2)


def _kuramoto_body(d1_ref, d2_ref, d4_ref, d8_ref, yt_ref, WpxT_ref, bpx_ref,
                   Wq_ref, Wk_ref, Wv_ref, Wo_ref, Gm_ref, WroS_ref, bro_ref,
                   xout_ref):
    d1 = d1_ref[...]
    d2 = d2_ref[...]
    d4 = d4_ref[...]
    d8 = d8_ref[...]
    x0 = bpx_ref[...][None, :]
    for g, blk in enumerate((d8, jnp.abs(d1 - d2), jnp.abs(d2 - d4),
                             jnp.abs(d4 - d8))):
        x0 = x0 + jax.lax.dot_general(
            blk, WpxT_ref[pl.ds(g * F, F), :], (((1,), (0,)), ((), ())),
            preferred_element_type=jnp.float32)
    Gm = Gm_ref[...]

    def gsum(v):
        return jax.lax.dot_general(v, Gm, (((1,), (0,)), ((), ())),
                                   preferred_element_type=jnp.float32)

    x = x0 * jax.lax.rsqrt(gsum(x0 * x0) + 1e-6)
    yt = yt_ref[...]
    scale = 1.0 / np.sqrt(DH)
    for _ in range(QSTEPS):
        Q = jax.lax.dot_general(x, Wq_ref[...], (((1,), (0,)), ((), ())),
                                preferred_element_type=jnp.float32)
        K = jax.lax.dot_general(x, Wk_ref[...], (((1,), (0,)), ((), ())),
                                preferred_element_type=jnp.float32)
        V = jax.lax.dot_general(x, Wv_ref[...], (((1,), (0,)), ((), ())),
                                preferred_element_type=jnp.float32)
        ones_col = jnp.ones((N, 1), jnp.float32)
        outs = []
        for h in range(H):
            Qh = Q[:, h * DH:(h + 1) * DH]
            Kh = K[:, h * DH:(h + 1) * DH]
            Vh1 = jnp.concatenate([V[:, h * DH:(h + 1) * DH], ones_col],
                                  axis=1)
            S = jax.lax.dot_general(Qh, Kh, (((1,), (1,)), ((), ())),
                                    preferred_element_type=jnp.float32) * scale
            Ex = jnp.exp(S)
            O2 = jax.lax.dot_general(Ex, Vh1, (((1,), (0,)), ((), ())),
                                     preferred_element_type=jnp.float32)
            outs.append(O2[:, :DH] / O2[:, DH:DH + 1])
        O = jnp.concatenate(outs, axis=1)
        Jx = jax.lax.dot_general(O, Wo_ref[...], (((1,), (0,)), ((), ())),
                                 preferred_element_type=jnp.float32)
        force = Jx + yt
        dot = gsum(force * x)
        xg = x + GAMMA * (force - dot * x)
        x = xg * jax.lax.rsqrt(gsum(xg * xg) + 1e-6)
    acc = jnp.full((N, CH), 1e-6, jnp.float32)
    for o in range(NOSC):
        zo = jax.lax.dot_general(x, WroS_ref[o], (((1,), (0,)), ((), ())),
                                 preferred_element_type=jnp.float32)
        acc = acc + zo * zo
    xout_ref[...] = jnp.sqrt(acc) + bro_ref[...][None, :]


def _kuramoto(d1, d2, d4, d8, yt, WpxT, bpx, Wq, Wk, Wv, Wo, Gm, WroS, bro):
    return pl.pallas_call(
        _kuramoto_body,
        out_shape=jax.ShapeDtypeStruct((N, CH), jnp.float32),
    )(d1, d2, d4, d8, yt, WpxT, bpx, Wq, Wk, Wv, Wo, Gm, WroS, bro)


def kernel(input, input_fc, input_sc, Wm, bm, Wpy, bpy, Wpx, bpx, Wq, Wk, Wv,
           Wo, Wro, bro, Wout, bout):
    del input_fc
    x = input[0]
    src = input_sc[0].astype(jnp.int32)
    dst = input_sc[1].astype(jnp.int32)

    y_t, logits = _encoder(x, Wm, bm, Wpy.T, bpy, Wout, bout)

    src2 = src.reshape(E // ECH, ECH)
    dst2 = dst.reshape(E // ECH, ECH)
    zout, _w = _sc_diffuse(x, src2, dst2)
    d1, d2, d4, d8 = zout[0], zout[1], zout[3], zout[7]

    Gm = jnp.repeat(jnp.repeat(jnp.eye(NG, dtype=jnp.float32), NOSC, axis=0),
                    NOSC, axis=1)
    WroS = jnp.stack([Wro[:, o::NOSC] for o in range(NOSC)], axis=0)
    x_out = _kuramoto(d1, d2, d4, d8, y_t, Wpx.T, bpx, Wq, Wk, Wv, Wo, Gm,
                      WroS, bro)

    logits_out = logits[None, :, :]
    x_out = x_out[None, :, :]
    saved_y = y_t.T[None, :, :]
    return logits_out, x_out, saved_y


# speedup vs baseline: 1.4077x; 1.0146x over previous
import functools
import jax
import jax.numpy as jnp
import numpy as np
from jax import lax
from jax.experimental import pallas as pl
from jax.experimental.pallas import tpu as pltpu
from jax.experimental.pallas import tpu_sc as plsc

N = 2048
F = 128
CH = 128
NOSC = 4
NG = CH // NOSC
H = 8
DH = CH // H
QSTEPS = 4
E = 32768
GST = 4
NCLS = 4
GAMMA = 1.0


def _encoder_body(x_ref, Wm_ref, bm_ref, WpyT_ref, bpy_ref, Wout_ref, bout_ref,
                  y_ref, logits_ref):
    x = x_ref[...]
    y = jnp.zeros((N, CH), jnp.float32) + bpy_ref[...][None, :]
    for k in range(GST):
        ys = jnp.maximum(
            jax.lax.dot_general(x, Wm_ref[k], (((1,), (0,)), ((), ())),
                                preferred_element_type=jnp.float32)
            + bm_ref[k][None, :], 0.0)
        y = y + jax.lax.dot_general(ys, WpyT_ref[pl.ds(k * F, F), :],
                                    (((1,), (0,)), ((), ())),
                                    preferred_element_type=jnp.float32)
    y_ref[...] = y
    logits_ref[...] = jax.lax.dot_general(
        y, Wout_ref[...], (((1,), (0,)), ((), ())),
        preferred_element_type=jnp.float32) + bout_ref[...][None, :]


def _encoder(x, Wm, bm, WpyT, bpy, Wout, bout):
    return pl.pallas_call(
        _encoder_body,
        out_shape=(jax.ShapeDtypeStruct((N, CH), jnp.float32),
                   jax.ShapeDtypeStruct((N, NCLS), jnp.float32)),
    )(x, Wm, bm, WpyT, bpy, Wout, bout)


TILES = 16
NPT = N // TILES
EPT = E // TILES
ECH = 128
NCHUNK = EPT // ECH


def _sc_diffuse_body(x0_hbm, src_hbm, dst_hbm, zout_hbm, w_hbm,
                     w_l, rows_a, rows_b, rows_c, zer_l,
                     src_l, dst_l, invdeg_l, acc_sh,
                     sem_a, sem_b, sem_c, sem_p):
    sid = lax.axis_index("s")
    base = sid * NPT
    ebase = sid * NCHUNK

    pltpu.sync_copy(src_hbm.at[pl.ds(ebase, NCHUNK)], src_l)
    pltpu.sync_copy(dst_hbm.at[pl.ds(ebase, NCHUNK)], dst_l)
    pltpu.sync_copy(x0_hbm.at[pl.ds(base, NPT)], w_l)

    zero16 = jnp.zeros((16,), jnp.float32)
    one16 = jnp.ones((16,), jnp.float32)

    def rows_loop(body):
        def outer(i, _):
            body(i)
            return 0
        lax.fori_loop(0, NPT, outer, 0)

    def fill(ref, val):
        def body(i):
            for j in range(F // 16):
                ref[i, pl.ds(j * 16, 16)] = val
        rows_loop(body)

    fill(zer_l, zero16)
    fill(rows_a, one16)

    pltpu.sync_copy(zer_l, acc_sh.at[pl.ds(base, NPT)])
    plsc.subcore_barrier()
    for c in range(NCHUNK):
        pltpu.sync_copy(rows_a, acc_sh.at[dst_l.at[c]], add=True)
    plsc.subcore_barrier()
    pltpu.sync_copy(acc_sh.at[pl.ds(base, NPT)], rows_b)

    def iv_slot(i):
        return (i // 8, pl.ds((i % 8) * 16, 16))

    def dv_body(i):
        d = rows_b[i, pl.ds(0, 16)]
        r, sl = iv_slot(i)
        invdeg_l[r, sl] = 1.0 / jnp.maximum(d, 1.0)
    rows_loop(dv_body)

    def scale_body(i):
        r, s0 = iv_slot(i)
        iv = invdeg_l[r, s0]
        for j in range(F // 16):
            sl = pl.ds(j * 16, 16)
            w_l[i, sl] = w_l[i, sl] * iv
    rows_loop(scale_body)

    def step_body(s, _):
        pub = pltpu.async_copy(w_l, w_hbm.at[pl.ds(base, NPT)], sem_p)
        pltpu.sync_copy(zer_l, acc_sh.at[pl.ds(base, NPT)])
        pub.wait()
        plsc.subcore_barrier()
        bufs = (rows_a, rows_b, rows_c)
        sems = (sem_a, sem_b, sem_c)
        descs = [None, None, None]
        descs[0] = pltpu.async_copy(w_hbm.at[dst_l.at[0]], rows_a, sem_a)
        descs[1] = pltpu.async_copy(w_hbm.at[dst_l.at[1]], rows_b, sem_b)
        for c in range(NCHUNK):
            descs[c % 3].wait()
            if c + 2 < NCHUNK:
                descs[(c + 2) % 3] = pltpu.async_copy(
                    w_hbm.at[dst_l.at[c + 2]], bufs[(c + 2) % 3],
                    sems[(c + 2) % 3])
            pltpu.sync_copy(bufs[c % 3], acc_sh.at[src_l.at[c]], add=True)
        plsc.subcore_barrier()
        pltpu.sync_copy(acc_sh.at[pl.ds(base, NPT)], rows_b)

        def upd_body(i):
            r, s0 = iv_slot(i)
            iv = invdeg_l[r, s0]
            for j in range(F // 16):
                sl = pl.ds(j * 16, 16)
                wv = 0.5 * w_l[i, sl] + (0.5 * iv) * rows_b[i, sl]
                w_l[i, sl] = wv
                rows_a[i, sl] = wv / iv
        rows_loop(upd_body)
        pltpu.sync_copy(rows_a, zout_hbm.at[s, pl.ds(base, NPT)])
        return 0
    lax.fori_loop(0, 8, step_body, 0)


def _sc_diffuse(x0, src2, dst2):
    mesh = plsc.VectorSubcoreMesh(core_axis_name="c", subcore_axis_name="s",
                                  num_cores=1)
    f = pl.kernel(
        _sc_diffuse_body, mesh=mesh,
        out_type=(jax.ShapeDtypeStruct((8, N, F), jnp.float32),
                  jax.ShapeDtypeStruct((N, F), jnp.float32)),
        scratch_types=[
            pltpu.VMEM((NPT, F), jnp.float32),
            pltpu.VMEM((NPT, F), jnp.float32),
            pltpu.VMEM((NPT, F), jnp.float32),
            pltpu.VMEM((NPT, F), jnp.float32),
            pltpu.VMEM((NPT, F), jnp.float32),
            pltpu.VMEM((NCHUNK, ECH), jnp.int32),
            pltpu.VMEM((NCHUNK, ECH), jnp.int32),
            pltpu.VMEM((NPT // 8, 128), jnp.float32),
            pltpu.VMEM_SHARED((N, F), jnp.float32),
            pltpu.SemaphoreType.DMA,
            pltpu.SemaphoreType.DMA,
            pltpu.SemaphoreType.DMA,
            pltpu.SemaphoreType.DMA,
        ])
    return f(x0, src2, dst2)


def _kuramoto_body(d1_ref, d2_ref, d4_ref, d8_ref, yt_ref, WpxT_ref, bpx_ref,
                   Wq_ref, Wk_ref, Wv_ref, Wo_ref, Gm_ref, WroS_ref, bro_ref,
                   xout_ref):
    d1 = d1_ref[...]
    d2 = d2_ref[...]
    d4 = d4_ref[...]
    d8 = d8_ref[...]
    x0 = bpx_ref[...][None, :]
    for g, blk in enumerate((d8, jnp.abs(d1 - d2), jnp.abs(d2 - d4),
                             jnp.abs(d4 - d8))):
        x0 = x0 + jax.lax.dot_general(
            blk, WpxT_ref[pl.ds(g * F, F), :], (((1,), (0,)), ((), ())),
            preferred_element_type=jnp.float32)
    Gm = Gm_ref[...]

    def gsum(v):
        return jax.lax.dot_general(v, Gm, (((1,), (0,)), ((), ())),
                                   preferred_element_type=jnp.float32)

    x = x0 * jax.lax.rsqrt(gsum(x0 * x0) + 1e-6)
    yt = yt_ref[...]
    scale = 1.0 / np.sqrt(DH)
    cidx = jax.lax.broadcasted_iota(jnp.int32, (H, 1, CH), 2) // DH
    hidx = jax.lax.broadcasted_iota(jnp.int32, (H, 1, CH), 0)
    maskE = jnp.where(cidx == hidx, 1.0, 0.0)
    blk8 = jnp.where(
        jax.lax.broadcasted_iota(jnp.int32, (H, 1, H), 2)
        == jax.lax.broadcasted_iota(jnp.int32, (H, 1, H), 0),
        1.0, 0.0)
    QB = 256
    for _ in range(QSTEPS):
        Q = jax.lax.dot_general(x, Wq_ref[...], (((1,), (0,)), ((), ())),
                                preferred_element_type=jnp.float32)
        K = jax.lax.dot_general(x, Wk_ref[...], (((1,), (0,)), ((), ())),
                                preferred_element_type=jnp.float32)
        V = jax.lax.dot_general(x, Wv_ref[...], (((1,), (0,)), ((), ())),
                                preferred_element_type=jnp.float32)
        Kexp = (jnp.broadcast_to(K[None, :, :] * scale, (H, N, CH))
                * maskE).reshape(H * N, CH)
        Vcat = jnp.concatenate(
            [jnp.broadcast_to(V[None, :, :], (H, N, CH)) * maskE,
             jnp.broadcast_to(blk8, (H, N, H))], axis=2).reshape(H * N,
                                                                 CH + H)
        jxs = []
        for qb in range(N // QB):
            Qb = Q[qb * QB:(qb + 1) * QB, :]
            Sb = jax.lax.dot_general(Qb, Kexp, (((1,), (1,)), ((), ())),
                                     preferred_element_type=jnp.float32)
            Eb = jnp.exp(Sb)
            O2 = jax.lax.dot_general(Eb, Vcat, (((1,), (0,)), ((), ())),
                                     preferred_element_type=jnp.float32)
            rec = 1.0 / O2[:, CH:CH + H]
            recE = jnp.broadcast_to(rec[:, :, None],
                                    (QB, H, DH)).reshape(QB, CH)
            jxs.append(O2[:, :CH] * recE)
        O = jnp.concatenate(jxs, axis=0)
        Jx = jax.lax.dot_general(O, Wo_ref[...], (((1,), (0,)), ((), ())),
                                 preferred_element_type=jnp.float32)
        force = Jx + yt
        dot = gsum(force * x)
        xg = x + GAMMA * (force - dot * x)
        x = xg * jax.lax.rsqrt(gsum(xg * xg) + 1e-6)
    acc = jnp.full((N, CH), 1e-6, jnp.float32)
    for o in range(NOSC):
        zo = jax.lax.dot_general(x, WroS_ref[o], (((1,), (0,)), ((), ())),
                                 preferred_element_type=jnp.float32)
        acc = acc + zo * zo
    xout_ref[...] = jnp.sqrt(acc) + bro_ref[...][None, :]


def _kuramoto(d1, d2, d4, d8, yt, WpxT, bpx, Wq, Wk, Wv, Wo, Gm, WroS, bro):
    return pl.pallas_call(
        _kuramoto_body,
        out_shape=jax.ShapeDtypeStruct((N, CH), jnp.float32),
    )(d1, d2, d4, d8, yt, WpxT, bpx, Wq, Wk, Wv, Wo, Gm, WroS, bro)


def kernel(input, input_fc, input_sc, Wm, bm, Wpy, bpy, Wpx, bpx, Wq, Wk, Wv,
           Wo, Wro, bro, Wout, bout):
    del input_fc
    x = input[0]
    src = input_sc[0].astype(jnp.int32)
    dst = input_sc[1].astype(jnp.int32)

    y_t, logits = _encoder(x, Wm, bm, Wpy.T, bpy, Wout, bout)

    src2 = src.reshape(E // ECH, ECH)
    dst2 = dst.reshape(E // ECH, ECH)
    zout, _w = _sc_diffuse(x, src2, dst2)
    d1, d2, d4, d8 = zout[0], zout[1], zout[3], zout[7]

    Gm = jnp.repeat(jnp.repeat(jnp.eye(NG, dtype=jnp.float32), NOSC, axis=0),
                    NOSC, axis=1)
    WroS = jnp.stack([Wro[:, o::NOSC] for o in range(NOSC)], axis=0)
    x_out = _kuramoto(d1, d2, d4, d8, y_t, Wpx.T, bpx, Wq, Wk, Wv, Wo, Gm,
                      WroS, bro)

    logits_out = logits[None, :, :]
    x_out = x_out[None, :, :]
    saved_y = y_t.T[None, :, :]
    return logits_out, x_out, saved_y


# speedup vs baseline: 1.4110x; 1.0023x over previous
import functools
import jax
import jax.numpy as jnp
import numpy as np
from jax import lax
from jax.experimental import pallas as pl
from jax.experimental.pallas import tpu as pltpu
from jax.experimental.pallas import tpu_sc as plsc

N = 2048
F = 128
CH = 128
NOSC = 4
NG = CH // NOSC
H = 8
DH = CH // H
QSTEPS = 4
E = 32768
GST = 4
NCLS = 4
GAMMA = 1.0


def _encoder_body(x_ref, Wm_ref, bm_ref, WpyT_ref, bpy_ref, Wout_ref, bout_ref,
                  y_ref, logits_ref):
    x = x_ref[...]
    y = jnp.zeros((N, CH), jnp.float32) + bpy_ref[...][None, :]
    for k in range(GST):
        ys = jnp.maximum(
            jax.lax.dot_general(x, Wm_ref[k], (((1,), (0,)), ((), ())),
                                preferred_element_type=jnp.float32)
            + bm_ref[k][None, :], 0.0)
        y = y + jax.lax.dot_general(ys, WpyT_ref[pl.ds(k * F, F), :],
                                    (((1,), (0,)), ((), ())),
                                    preferred_element_type=jnp.float32)
    y_ref[...] = y
    logits_ref[...] = jax.lax.dot_general(
        y, Wout_ref[...], (((1,), (0,)), ((), ())),
        preferred_element_type=jnp.float32) + bout_ref[...][None, :]


def _encoder(x, Wm, bm, WpyT, bpy, Wout, bout):
    return pl.pallas_call(
        _encoder_body,
        out_shape=(jax.ShapeDtypeStruct((N, CH), jnp.float32),
                   jax.ShapeDtypeStruct((N, NCLS), jnp.float32)),
    )(x, Wm, bm, WpyT, bpy, Wout, bout)


TILES = 16
NPT = N // TILES
EPT = E // TILES
ECH = 128
NCHUNK = EPT // ECH


def _sc_diffuse_body(x0_hbm, src_hbm, dst_hbm, zout_hbm, w_hbm,
                     w_l, rows_a, rows_b, rows_c, zer_l,
                     src_l, dst_l, invdeg_l, acc_sh,
                     sem_a, sem_b, sem_c, sem_p):
    sid = lax.axis_index("s")
    base = sid * NPT
    ebase = sid * NCHUNK

    pltpu.sync_copy(src_hbm.at[pl.ds(ebase, NCHUNK)], src_l)
    pltpu.sync_copy(dst_hbm.at[pl.ds(ebase, NCHUNK)], dst_l)
    pltpu.sync_copy(x0_hbm.at[pl.ds(base, NPT)], w_l)

    zero16 = jnp.zeros((16,), jnp.float32)
    one16 = jnp.ones((16,), jnp.float32)

    def rows_loop(body):
        def outer(i, _):
            body(i)
            return 0
        lax.fori_loop(0, NPT, outer, 0)

    def fill(ref, val):
        def body(i):
            for j in range(F // 16):
                ref[i, pl.ds(j * 16, 16)] = val
        rows_loop(body)

    fill(zer_l, zero16)
    fill(rows_a, one16)

    pltpu.sync_copy(zer_l, acc_sh.at[pl.ds(base, NPT)])
    plsc.subcore_barrier()
    for c in range(NCHUNK):
        pltpu.sync_copy(rows_a, acc_sh.at[dst_l.at[c]], add=True)
    plsc.subcore_barrier()
    pltpu.sync_copy(acc_sh.at[pl.ds(base, NPT)], rows_b)

    def iv_slot(i):
        return (i // 8, pl.ds((i % 8) * 16, 16))

    def dv_body(i):
        d = rows_b[i, pl.ds(0, 16)]
        r, sl = iv_slot(i)
        invdeg_l[r, sl] = 1.0 / jnp.maximum(d, 1.0)
    rows_loop(dv_body)

    def scale_body(i):
        r, s0 = iv_slot(i)
        iv = invdeg_l[r, s0]
        for j in range(F // 16):
            sl = pl.ds(j * 16, 16)
            w_l[i, sl] = w_l[i, sl] * iv
    rows_loop(scale_body)

    def step_body(s, _):
        pub = pltpu.async_copy(w_l, w_hbm.at[pl.ds(base, NPT)], sem_p)
        pltpu.sync_copy(zer_l, acc_sh.at[pl.ds(base, NPT)])
        pub.wait()
        plsc.subcore_barrier()
        bufs = (rows_a, rows_b, rows_c)
        gsems = (sem_a, sem_b, sem_c)
        gds = [None, None, None]
        gds[0] = pltpu.async_copy(w_hbm.at[dst_l.at[0]], rows_a, sem_a)
        gds[1] = pltpu.async_copy(w_hbm.at[dst_l.at[1]], rows_b, sem_b)
        for c in range(NCHUNK):
            gds[c % 3].wait()
            if c + 2 < NCHUNK:
                gds[(c + 2) % 3] = pltpu.async_copy(
                    w_hbm.at[dst_l.at[c + 2]], bufs[(c + 2) % 3],
                    gsems[(c + 2) % 3])
            pltpu.sync_copy(bufs[c % 3], acc_sh.at[src_l.at[c]], add=True)
        plsc.subcore_barrier()
        pltpu.sync_copy(acc_sh.at[pl.ds(base, NPT)], rows_b)

        def upd_body(i):
            r, s0 = iv_slot(i)
            iv = invdeg_l[r, s0]
            for j in range(F // 16):
                sl = pl.ds(j * 16, 16)
                wv = 0.5 * w_l[i, sl] + (0.5 * iv) * rows_b[i, sl]
                w_l[i, sl] = wv
                rows_a[i, sl] = wv / iv
        rows_loop(upd_body)
        pltpu.sync_copy(rows_a, zout_hbm.at[s, pl.ds(base, NPT)])
        return 0
    lax.fori_loop(0, 8, step_body, 0)


def _sc_diffuse(x0, src2, dst2):
    mesh = plsc.VectorSubcoreMesh(core_axis_name="c", subcore_axis_name="s",
                                  num_cores=1)
    f = pl.kernel(
        _sc_diffuse_body, mesh=mesh,
        out_type=(jax.ShapeDtypeStruct((8, N, F), jnp.float32),
                  jax.ShapeDtypeStruct((N, F), jnp.float32)),
        scratch_types=[
            pltpu.VMEM((NPT, F), jnp.float32),
            pltpu.VMEM((NPT, F), jnp.float32),
            pltpu.VMEM((NPT, F), jnp.float32),
            pltpu.VMEM((NPT, F), jnp.float32),
            pltpu.VMEM((NPT, F), jnp.float32),
            pltpu.VMEM((NCHUNK, ECH), jnp.int32),
            pltpu.VMEM((NCHUNK, ECH), jnp.int32),
            pltpu.VMEM((NPT // 8, 128), jnp.float32),
            pltpu.VMEM_SHARED((N, F), jnp.float32),
            pltpu.SemaphoreType.DMA,
            pltpu.SemaphoreType.DMA,
            pltpu.SemaphoreType.DMA,
            pltpu.SemaphoreType.DMA,
        ])
    return f(x0, src2, dst2)


def _kuramoto_body(d1_ref, d2_ref, d4_ref, d8_ref, yt_ref, WpxT_ref, bpx_ref,
                   Wq_ref, Wk_ref, Wv_ref, Wo_ref, Gm_ref, WroS_ref, bro_ref,
                   xout_ref):
    d1 = d1_ref[...]
    d2 = d2_ref[...]
    d4 = d4_ref[...]
    d8 = d8_ref[...]
    x0 = bpx_ref[...][None, :]
    for g, blk in enumerate((d8, jnp.abs(d1 - d2), jnp.abs(d2 - d4),
                             jnp.abs(d4 - d8))):
        x0 = x0 + jax.lax.dot_general(
            blk, WpxT_ref[pl.ds(g * F, F), :], (((1,), (0,)), ((), ())),
            preferred_element_type=jnp.float32)
    Gm = Gm_ref[...]

    def gsum(v):
        return jax.lax.dot_general(v, Gm, (((1,), (0,)), ((), ())),
                                   preferred_element_type=jnp.float32)

    x = x0 * jax.lax.rsqrt(gsum(x0 * x0) + 1e-6)
    yt = yt_ref[...]
    scale = 1.0 / np.sqrt(DH)
    cidx = jax.lax.broadcasted_iota(jnp.int32, (H, 1, CH), 2) // DH
    hidx = jax.lax.broadcasted_iota(jnp.int32, (H, 1, CH), 0)
    maskE = jnp.where(cidx == hidx, 1.0, 0.0)
    blk8 = jnp.where(
        jax.lax.broadcasted_iota(jnp.int32, (H, 1, H), 2)
        == jax.lax.broadcasted_iota(jnp.int32, (H, 1, H), 0),
        1.0, 0.0)
    QB = 256
    for _ in range(QSTEPS):
        Q = jax.lax.dot_general(x, Wq_ref[...], (((1,), (0,)), ((), ())),
                                preferred_element_type=jnp.float32)
        K = jax.lax.dot_general(x, Wk_ref[...], (((1,), (0,)), ((), ())),
                                preferred_element_type=jnp.float32)
        V = jax.lax.dot_general(x, Wv_ref[...], (((1,), (0,)), ((), ())),
                                preferred_element_type=jnp.float32)
        Kexp = (jnp.broadcast_to(K[None, :, :] * scale, (H, N, CH))
                * maskE).reshape(H * N, CH)
        Vcat = jnp.concatenate(
            [jnp.broadcast_to(V[None, :, :], (H, N, CH)) * maskE,
             jnp.broadcast_to(blk8, (H, N, H))], axis=2).reshape(H * N,
                                                                 CH + H)
        jxs = []
        for qb in range(N // QB):
            Qb = Q[qb * QB:(qb + 1) * QB, :]
            Sb = jax.lax.dot_general(Qb, Kexp, (((1,), (1,)), ((), ())),
                                     preferred_element_type=jnp.float32)
            Eb = jnp.exp(Sb)
            O2 = jax.lax.dot_general(Eb, Vcat, (((1,), (0,)), ((), ())),
                                     preferred_element_type=jnp.float32)
            rec = 1.0 / O2[:, CH:CH + H]
            recE = jnp.broadcast_to(rec[:, :, None],
                                    (QB, H, DH)).reshape(QB, CH)
            jxs.append(O2[:, :CH] * recE)
        O = jnp.concatenate(jxs, axis=0)
        Jx = jax.lax.dot_general(O, Wo_ref[...], (((1,), (0,)), ((), ())),
                                 preferred_element_type=jnp.float32)
        force = Jx + yt
        dot = gsum(force * x)
        xg = x + GAMMA * (force - dot * x)
        x = xg * jax.lax.rsqrt(gsum(xg * xg) + 1e-6)
    acc = jnp.full((N, CH), 1e-6, jnp.float32)
    for o in range(NOSC):
        zo = jax.lax.dot_general(x, WroS_ref[o], (((1,), (0,)), ((), ())),
                                 preferred_element_type=jnp.float32)
        acc = acc + zo * zo
    xout_ref[...] = jnp.sqrt(acc) + bro_ref[...][None, :]


def _kuramoto(d1, d2, d4, d8, yt, WpxT, bpx, Wq, Wk, Wv, Wo, Gm, WroS, bro):
    return pl.pallas_call(
        _kuramoto_body,
        out_shape=jax.ShapeDtypeStruct((N, CH), jnp.float32),
    )(d1, d2, d4, d8, yt, WpxT, bpx, Wq, Wk, Wv, Wo, Gm, WroS, bro)


def kernel(input, input_fc, input_sc, Wm, bm, Wpy, bpy, Wpx, bpx, Wq, Wk, Wv,
           Wo, Wro, bro, Wout, bout):
    del input_fc
    x = input[0]
    src = input_sc[0].astype(jnp.int32)
    dst = input_sc[1].astype(jnp.int32)

    y_t, logits = _encoder(x, Wm, bm, Wpy.T, bpy, Wout, bout)

    src2 = src.reshape(E // ECH, ECH)
    dst2 = dst.reshape(E // ECH, ECH)
    zout, _w = _sc_diffuse(x, src2, dst2)
    d1, d2, d4, d8 = zout[0], zout[1], zout[3], zout[7]

    Gm = jnp.repeat(jnp.repeat(jnp.eye(NG, dtype=jnp.float32), NOSC, axis=0),
                    NOSC, axis=1)
    WroS = jnp.stack([Wro[:, o::NOSC] for o in range(NOSC)], axis=0)
    x_out = _kuramoto(d1, d2, d4, d8, y_t, Wpx.T, bpx, Wq, Wk, Wv, Wo, Gm,
                      WroS, bro)

    logits_out = logits[None, :, :]
    x_out = x_out[None, :, :]
    saved_y = y_t.T[None, :, :]
    return logits_out, x_out, saved_y
